# Initial kernel scaffold; baseline (speedup 1.0000x reference)
import jax
import jax.numpy as jnp
from jax.experimental import pallas as pl


def kernel(feature_emb, edge_index, W_out0, W_in0, bias0, W_out1, W_in1, bias1, W_ih, W_hh, b_ih, b_hh):
    raise NotImplementedError("write your pallas kernel here")


# speedup vs baseline: 3.9383x; 3.9383x over previous
import functools

import jax
import jax.numpy as jnp
from jax import lax
from jax.experimental import pallas as pl
from jax.experimental.pallas import tpu as pltpu
from jax.experimental.pallas import tpu_sc as plsc

F = 2048
D = 64
E = 65504
NC = 2
NS = 16
NW = NC * NS
EPAD = 65536
EW = EPAD // NW
CH = 128
NCH = EW // CH
VROWS = F + 128
DD = 2 * D
RPT = VROWS // NS


def _sc_segment_sum(w_cat, src_idx, dst_idx, zeros):
    mesh = plsc.VectorSubcoreMesh(core_axis_name="c", subcore_axis_name="s")

    @functools.partial(
        pl.kernel,
        out_type=jax.ShapeDtypeStruct((NC, VROWS, DD), jnp.float32),
        mesh=mesh,
        scratch_types=[
            pltpu.VMEM((NCH, CH), jnp.int32),
            pltpu.VMEM((NCH, CH), jnp.int32),
            pltpu.VMEM((CH, DD), jnp.float32),
            pltpu.VMEM((CH, DD), jnp.float32),
            pltpu.VMEM_SHARED((VROWS, DD), jnp.float32),
            pltpu.SemaphoreType.DMA,
            pltpu.SemaphoreType.DMA,
        ],
    )
    def seg(w_hbm, src_hbm, dst_hbm, z_hbm, out_hbm,
            src_v, dst_v, buf0, buf1, v_sh, sem0, sem1):
        c = lax.axis_index("c")
        s = lax.axis_index("s")
        wid = c * NS + s

        pltpu.sync_copy(z_hbm.at[pl.ds(s * RPT, RPT)],
                        v_sh.at[pl.ds(s * RPT, RPT)])
        pltpu.sync_copy(src_hbm.at[wid], src_v)
        pltpu.sync_copy(dst_hbm.at[wid], dst_v)
        plsc.subcore_barrier()

        bufs = (buf0, buf1)
        sems = (sem0, sem1)
        cp0 = pltpu.make_async_copy(w_hbm.at[src_v.at[0]], bufs[0], sems[0])
        cp0.start()
        for j in range(NCH):
            if j + 1 < NCH:
                nxt = pltpu.make_async_copy(
                    w_hbm.at[src_v.at[j + 1]], bufs[(j + 1) % 2],
                    sems[(j + 1) % 2])
                nxt.start()
            pltpu.make_async_copy(
                w_hbm.at[src_v.at[j]], bufs[j % 2], sems[j % 2]).wait()
            pltpu.sync_copy(bufs[j % 2], v_sh.at[dst_v.at[j]], add=True)

        plsc.subcore_barrier()
        pltpu.sync_copy(v_sh.at[pl.ds(s * RPT, RPT)],
                        out_hbm.at[c, pl.ds(s * RPT, RPT)])

    return seg(w_cat, src_idx, dst_idx, zeros)


def _tc_body(v0_ref, v1_ref, wout0_ref, wout1_ref, femb_ref,
             wr_ref, wz_ref, wn_ref, ur_ref, uz_ref, un_ref,
             br_ref, bz_ref, bn_ref, cr_ref, cz_ref, cn_ref,
             b0_ref, b1_ref, out_ref):
    v0 = v0_ref[0] + v0_ref[1]
    v1 = v1_ref[0] + v1_ref[1]
    dn = (((0,), (0,)), ((), ()))
    m0 = lax.dot_general(wout0_ref[...], v0, dn,
                         preferred_element_type=jnp.float32)
    m1 = lax.dot_general(wout1_ref[...], v1, dn,
                         preferred_element_type=jnp.float32)
    femb = femb_ref[...]
    wr, wz, wn = wr_ref[...], wz_ref[...], wn_ref[...]
    ur, uz, un = ur_ref[...], uz_ref[...], un_ref[...]

    def mm(x, w):
        return lax.dot_general(x, w, (((1,), (1,)), ((), ())),
                               preferred_element_type=jnp.float32)

    h = femb
    for m, b_ref in ((m0, b0_ref), (m1, b1_ref)):
        a = jnp.dot(h, m, preferred_element_type=jnp.float32) + b_ref[...]
        r = jax.nn.sigmoid(mm(a, wr) + br_ref[...] + mm(h, ur) + cr_ref[...])
        z = jax.nn.sigmoid(mm(a, wz) + bz_ref[...] + mm(h, uz) + cz_ref[...])
        n = jnp.tanh(mm(a, wn) + bn_ref[...] + r * (mm(h, un) + cn_ref[...]))
        h = (1.0 - z) * n + z * h + femb
    out_ref[...] = h


def kernel(feature_emb, edge_index, W_out0, W_in0, bias0,
           W_out1, W_in1, bias1, W_ih, W_hh, b_ih, b_hh):
    w_cat = jnp.concatenate([W_in0, W_in1], axis=1)
    src = edge_index[0].astype(jnp.int32)
    dst = edge_index[1].astype(jnp.int32)
    pad = EPAD - E
    src_p = jnp.concatenate([src, jnp.zeros((pad,), jnp.int32)])
    dst_p = jnp.concatenate([dst, jnp.full((pad,), F, jnp.int32)])
    src_p = src_p.reshape(NW, NCH, CH)
    dst_p = dst_p.reshape(NW, NCH, CH)
    zeros = jnp.zeros((VROWS, DD), jnp.float32)

    vpart = _sc_segment_sum(w_cat, src_p, dst_p, zeros)
    v0 = vpart[:, :F, :D]
    v1 = vpart[:, :F, D:]

    wr, wz, wn = W_ih[:D], W_ih[D:2 * D], W_ih[2 * D:]
    ur, uz, un = W_hh[:D], W_hh[D:2 * D], W_hh[2 * D:]
    br, bz, bn = (b_ih[:D].reshape(1, D), b_ih[D:2 * D].reshape(1, D),
                  b_ih[2 * D:].reshape(1, D))
    cr, cz, cn = (b_hh[:D].reshape(1, D), b_hh[D:2 * D].reshape(1, D),
                  b_hh[2 * D:].reshape(1, D))

    return pl.pallas_call(
        _tc_body,
        out_shape=jax.ShapeDtypeStruct((F, D), jnp.float32),
    )(v0, v1, W_out0, W_out1, feature_emb,
      wr, wz, wn, ur, uz, un,
      br, bz, bn, cr, cz, cn,
      bias0.reshape(1, D), bias1.reshape(1, D))


# speedup vs baseline: 3.9983x; 1.0152x over previous
import functools

import jax
import jax.numpy as jnp
from jax import lax
from jax.experimental import pallas as pl
from jax.experimental.pallas import tpu as pltpu
from jax.experimental.pallas import tpu_sc as plsc

F = 2048
D = 64
E = 65504
NC = 2
NS = 16
NW = NC * NS
EW = 2048
CH = 128
NCH = EW // CH
VROWS = F + 128
DD = 2 * D
RPT = VROWS // NS
TAIL = NW * EW - E
REAL_TAIL = CH - TAIL


def _sc_segment_sum(w_cat, eidx_flat, zeros):
    mesh = plsc.VectorSubcoreMesh(core_axis_name="c", subcore_axis_name="s")

    @functools.partial(
        pl.kernel,
        out_type=jax.ShapeDtypeStruct((NC, VROWS, DD), jnp.float32),
        mesh=mesh,
        scratch_types=[
            pltpu.VMEM((EW,), jnp.int32),
            pltpu.VMEM((NCH, CH), jnp.int32),
            pltpu.VMEM((CH, DD), jnp.float32),
            pltpu.VMEM((CH, DD), jnp.float32),
            pltpu.VMEM_SHARED((VROWS, DD), jnp.float32),
            pltpu.SemaphoreType.DMA,
            pltpu.SemaphoreType.DMA,
            pltpu.SemaphoreType.DMA,
        ],
    )
    def seg(w_hbm, e_hbm, z_hbm, out_hbm,
            src_v, dst_v, buf0, buf1, v_sh, sem0, sem1, semi):
        c = lax.axis_index("c")
        s = lax.axis_index("s")
        wid = c * NS + s
        base = wid * EW

        zcp = pltpu.make_async_copy(z_hbm, v_sh.at[pl.ds(s * RPT, RPT)], semi)
        zcp.start()

        last = wid == NW - 1

        @pl.when(jnp.logical_not(last))
        def _():
            pltpu.sync_copy(e_hbm.at[pl.ds(base, EW)], src_v)
            for j in range(NCH):
                pltpu.sync_copy(e_hbm.at[pl.ds(E + base + j * CH, CH)],
                                dst_v.at[j])

        @pl.when(last)
        def _():
            pltpu.sync_copy(e_hbm.at[pl.ds(base, EW - TAIL)],
                            src_v.at[pl.ds(0, EW - TAIL)])
            for j in range(NCH - 1):
                pltpu.sync_copy(e_hbm.at[pl.ds(E + base + j * CH, CH)],
                                dst_v.at[j])
            pltpu.sync_copy(
                e_hbm.at[pl.ds(E + base + (NCH - 1) * CH, REAL_TAIL)],
                dst_v.at[NCH - 1, pl.ds(0, REAL_TAIL)])
            for t in range(REAL_TAIL, CH, 16):
                src_v[pl.ds(EW - TAIL + t - REAL_TAIL, 16)] = jnp.zeros(
                    (16,), jnp.int32)
                dst_v[NCH - 1, pl.ds(t, 16)] = jnp.full((16,), F, jnp.int32)

        zcp.wait()
        plsc.subcore_barrier()

        bufs = (buf0, buf1)
        sems = (sem0, sem1)
        pltpu.make_async_copy(
            w_hbm.at[src_v.at[pl.ds(0, CH)]], bufs[0], sems[0]).start()
        for j in range(NCH):
            if j + 1 < NCH:
                pltpu.make_async_copy(
                    w_hbm.at[src_v.at[pl.ds((j + 1) * CH, CH)]],
                    bufs[(j + 1) % 2], sems[(j + 1) % 2]).start()
            pltpu.make_async_copy(
                w_hbm.at[src_v.at[pl.ds(j * CH, CH)]],
                bufs[j % 2], sems[j % 2]).wait()
            pltpu.sync_copy(bufs[j % 2], v_sh.at[dst_v.at[j]], add=True)

        plsc.subcore_barrier()
        pltpu.sync_copy(v_sh.at[pl.ds(s * RPT, RPT)],
                        out_hbm.at[c, pl.ds(s * RPT, RPT)])

    return seg(w_cat, eidx_flat, zeros)


def _tc_body(vpart_ref, wout0_ref, wout1_ref, femb_ref,
             wr_ref, wz_ref, wn_ref, ur_ref, uz_ref, un_ref,
             br_ref, bz_ref, bn_ref, cr_ref, cz_ref, cn_ref,
             b0_ref, b1_ref, out_ref):
    vsum = vpart_ref[0, :F, :] + vpart_ref[1, :F, :]
    wcat = jnp.concatenate([wout0_ref[...], wout1_ref[...]], axis=1)
    dn = (((0,), (0,)), ((), ()))
    x = lax.dot_general(wcat, vsum, dn,
                        preferred_element_type=jnp.float32)
    m0 = x[:D, :D]
    m1 = x[D:, D:]
    femb = femb_ref[...]
    wr, wz, wn = wr_ref[...], wz_ref[...], wn_ref[...]
    ur, uz, un = ur_ref[...], uz_ref[...], un_ref[...]

    def mm(a, w):
        return lax.dot_general(a, w, (((1,), (1,)), ((), ())),
                               preferred_element_type=jnp.float32)

    h = femb
    for m, b_ref in ((m0, b0_ref), (m1, b1_ref)):
        a = jnp.dot(h, m, preferred_element_type=jnp.float32) + b_ref[...]
        r = jax.nn.sigmoid(mm(a, wr) + br_ref[...] + mm(h, ur) + cr_ref[...])
        z = jax.nn.sigmoid(mm(a, wz) + bz_ref[...] + mm(h, uz) + cz_ref[...])
        n = jnp.tanh(mm(a, wn) + bn_ref[...] + r * (mm(h, un) + cn_ref[...]))
        h = (1.0 - z) * n + z * h + femb
    out_ref[...] = h


def kernel(feature_emb, edge_index, W_out0, W_in0, bias0,
           W_out1, W_in1, bias1, W_ih, W_hh, b_ih, b_hh):
    w_cat = jnp.concatenate([W_in0, W_in1], axis=1)
    eidx_flat = edge_index.astype(jnp.int32).reshape(-1)
    zeros = jnp.zeros((RPT, DD), jnp.float32)

    vpart = _sc_segment_sum(w_cat, eidx_flat, zeros)

    wr, wz, wn = W_ih[:D], W_ih[D:2 * D], W_ih[2 * D:]
    ur, uz, un = W_hh[:D], W_hh[D:2 * D], W_hh[2 * D:]
    br, bz, bn = (b_ih[:D].reshape(1, D), b_ih[D:2 * D].reshape(1, D),
                  b_ih[2 * D:].reshape(1, D))
    cr, cz, cn = (b_hh[:D].reshape(1, D), b_hh[D:2 * D].reshape(1, D),
                  b_hh[2 * D:].reshape(1, D))

    return pl.pallas_call(
        _tc_body,
        out_shape=jax.ShapeDtypeStruct((F, D), jnp.float32),
    )(vpart, W_out0, W_out1, feature_emb,
      wr, wz, wn, ur, uz, un,
      br, bz, bn, cr, cz, cn,
      bias0.reshape(1, D), bias1.reshape(1, D))


# speedup vs baseline: 4.3020x; 1.0759x over previous
import functools

import jax
import jax.numpy as jnp
from jax import lax
from jax.experimental import pallas as pl
from jax.experimental.pallas import tpu as pltpu
from jax.experimental.pallas import tpu_sc as plsc

F = 2048
D = 64
E = 65504
NC = 2
NS = 16
NW = NC * NS
EW = 2048
CH = 128
NCH = EW // CH
VROWS = F + 128
DD = 2 * D
RPT = VROWS // NS
TAIL = NW * EW - E
REAL_TAIL = CH - TAIL


def _sc_segment_sum(w_cat, eidx_flat, zeros):
    mesh = plsc.VectorSubcoreMesh(core_axis_name="c", subcore_axis_name="s")

    @functools.partial(
        pl.kernel,
        out_type=jax.ShapeDtypeStruct((NC, VROWS, DD), jnp.float32),
        mesh=mesh,
        scratch_types=[
            pltpu.VMEM((EW,), jnp.int32),
            pltpu.VMEM((EW,), jnp.int32),
            pltpu.VMEM((CH, DD), jnp.float32),
            pltpu.VMEM((CH, DD), jnp.float32),
            pltpu.VMEM_SHARED((VROWS, DD), jnp.float32),
            pltpu.SemaphoreType.DMA,
            pltpu.SemaphoreType.DMA,
            pltpu.SemaphoreType.DMA,
        ],
    )
    def seg(w_hbm, e_hbm, z_hbm, out_hbm,
            src_v, dst_v, buf0, buf1, v_sh, sem0, sem1, semi):
        c = lax.axis_index("c")
        s = lax.axis_index("s")
        wid = c * NS + s
        base = wid * EW

        zcp = pltpu.make_async_copy(z_hbm, v_sh.at[pl.ds(s * RPT, RPT)], semi)
        zcp.start()

        last = wid == NW - 1

        @pl.when(jnp.logical_not(last))
        def _():
            pltpu.sync_copy(e_hbm.at[pl.ds(base, EW)], src_v)
            pltpu.sync_copy(e_hbm.at[pl.ds(E + base, EW)], dst_v)

        @pl.when(last)
        def _():
            pltpu.sync_copy(e_hbm.at[pl.ds(base, EW - TAIL)],
                            src_v.at[pl.ds(0, EW - TAIL)])
            pltpu.sync_copy(e_hbm.at[pl.ds(E + base, EW - TAIL)],
                            dst_v.at[pl.ds(0, EW - TAIL)])
            for t in range(EW - TAIL, EW, 16):
                src_v[pl.ds(t, 16)] = jnp.zeros((16,), jnp.int32)
                dst_v[pl.ds(t, 16)] = jnp.full((16,), F, jnp.int32)

        zcp.wait()
        plsc.subcore_barrier()

        bufs = (buf0, buf1)
        sems = (sem0, sem1)
        pltpu.make_async_copy(
            w_hbm.at[src_v.at[pl.ds(0, CH)]], bufs[0], sems[0]).start()
        for j in range(NCH):
            if j + 1 < NCH:
                pltpu.make_async_copy(
                    w_hbm.at[src_v.at[pl.ds((j + 1) * CH, CH)]],
                    bufs[(j + 1) % 2], sems[(j + 1) % 2]).start()
            pltpu.make_async_copy(
                w_hbm.at[src_v.at[pl.ds(j * CH, CH)]],
                bufs[j % 2], sems[j % 2]).wait()
            pltpu.sync_copy(bufs[j % 2],
                            v_sh.at[dst_v.at[pl.ds(j * CH, CH)]], add=True)

        plsc.subcore_barrier()
        pltpu.sync_copy(v_sh.at[pl.ds(s * RPT, RPT)],
                        out_hbm.at[c, pl.ds(s * RPT, RPT)])

    return seg(w_cat, eidx_flat, zeros)


def _tc_body(vpart_ref, wout0_ref, wout1_ref, femb_ref,
             wr_ref, wz_ref, wn_ref, ur_ref, uz_ref, un_ref,
             br_ref, bz_ref, bn_ref, cr_ref, cz_ref, cn_ref,
             b0_ref, b1_ref, out_ref):
    vsum = vpart_ref[0, :F, :] + vpart_ref[1, :F, :]
    wcat = jnp.concatenate([wout0_ref[...], wout1_ref[...]], axis=1)
    dn = (((0,), (0,)), ((), ()))
    x = lax.dot_general(wcat, vsum, dn,
                        preferred_element_type=jnp.float32)
    m0 = x[:D, :D]
    m1 = x[D:, D:]
    femb = femb_ref[...]
    wr, wz, wn = wr_ref[...], wz_ref[...], wn_ref[...]
    ur, uz, un = ur_ref[...], uz_ref[...], un_ref[...]

    def mm(a, w):
        return lax.dot_general(a, w, (((1,), (1,)), ((), ())),
                               preferred_element_type=jnp.float32)

    h = femb
    for m, b_ref in ((m0, b0_ref), (m1, b1_ref)):
        a = jnp.dot(h, m, preferred_element_type=jnp.float32) + b_ref[...]
        r = jax.nn.sigmoid(mm(a, wr) + br_ref[...] + mm(h, ur) + cr_ref[...])
        z = jax.nn.sigmoid(mm(a, wz) + bz_ref[...] + mm(h, uz) + cz_ref[...])
        n = jnp.tanh(mm(a, wn) + bn_ref[...] + r * (mm(h, un) + cn_ref[...]))
        h = (1.0 - z) * n + z * h + femb
    out_ref[...] = h


def kernel(feature_emb, edge_index, W_out0, W_in0, bias0,
           W_out1, W_in1, bias1, W_ih, W_hh, b_ih, b_hh):
    w_cat = jnp.concatenate([W_in0, W_in1], axis=1)
    eidx_flat = edge_index.astype(jnp.int32).reshape(-1)
    zeros = jnp.zeros((RPT, DD), jnp.float32)

    vpart = _sc_segment_sum(w_cat, eidx_flat, zeros)

    wr, wz, wn = W_ih[:D], W_ih[D:2 * D], W_ih[2 * D:]
    ur, uz, un = W_hh[:D], W_hh[D:2 * D], W_hh[2 * D:]
    br, bz, bn = (b_ih[:D].reshape(1, D), b_ih[D:2 * D].reshape(1, D),
                  b_ih[2 * D:].reshape(1, D))
    cr, cz, cn = (b_hh[:D].reshape(1, D), b_hh[D:2 * D].reshape(1, D),
                  b_hh[2 * D:].reshape(1, D))

    return pl.pallas_call(
        _tc_body,
        out_shape=jax.ShapeDtypeStruct((F, D), jnp.float32),
    )(vpart, W_out0, W_out1, feature_emb,
      wr, wz, wn, ur, uz, un,
      br, bz, bn, cr, cz, cn,
      bias0.reshape(1, D), bias1.reshape(1, D))
